# baseline (device time: 104153 ns/iter reference)
import jax
import jax.numpy as jnp
from jax import lax
from jax.experimental import pallas as pl
from jax.experimental.pallas import tpu as pltpu

N_DEV = 4
SCALE = 0.08838834764831843
SQ = 256
SKV = 4096
HQ = 8
DH = 128
NB = 4
BQ = 64
GK = SKV // (NB * BQ)
KV_R = GK * BQ
D_MODEL = HQ * DH


def kernel(x, Wq, K_ext, V_ext, Wo):
    x2 = x.reshape(SQ, D_MODEL)
    K4 = K_ext.reshape(GK, NB, BQ, D_MODEL).astype(jnp.bfloat16)
    V4 = V_ext.reshape(GK, NB, BQ, D_MODEL).astype(jnp.bfloat16)

    def body(x_ref, wq_ref, k_ref, v_ref, wo_ref, out_ref,
             qbuf, olbuf, lbuf, pme_o, pme_l, ptmp_o, ptmp_l,
             kp_ref, vp_ref,
             qs_sems, qr_sems, os_sems, or_sems, ls_sems, lr_sems,
             kv_sems):
        my = lax.axis_index("i")
        left = (my + N_DEV - 1) % N_DEV
        right = (my + 1) % N_DEV

        kv_copies = []
        for r in range(NB):
            c = pltpu.make_async_copy(k_ref.at[:, r], kp_ref.at[r],
                                      kv_sems.at[r])
            c.start()
            kv_copies.append(c)
            c = pltpu.make_async_copy(v_ref.at[:, r], vp_ref.at[r],
                                      kv_sems.at[NB + r])
            c.start()
            kv_copies.append(c)

        barrier_sem = pltpu.get_barrier_semaphore()
        for nbr in (left, right):
            pl.semaphore_signal(barrier_sem, inc=1, device_id=(nbr,),
                                device_id_type=pl.DeviceIdType.MESH)
        pl.semaphore_wait(barrier_sem, 2)

        def ring(buf, ssems, rsems, k):
            return pltpu.make_async_remote_copy(
                src_ref=buf.at[k], dst_ref=buf.at[k + 1],
                send_sem=ssems.at[k], recv_sem=rsems.at[k],
                device_id=(right,), device_id_type=pl.DeviceIdType.MESH)

        qsend = [ring(qbuf, qs_sems, qr_sems, k) for k in range(N_DEV - 1)]
        osend = [ring(olbuf, os_sems, or_sems, k) for k in range(N_DEV - 1)]
        lsend = [ring(lbuf, ls_sems, lr_sems, k) for k in range(N_DEV - 1)]

        q = jnp.dot(x_ref[...].astype(jnp.bfloat16),
                    wq_ref[...].astype(jnp.bfloat16),
                    preferred_element_type=jnp.float32) * SCALE
        qbuf[0, :, :] = q.astype(jnp.bfloat16)
        qsend[0].start()

        for c in kv_copies:
            c.wait()

        def partial(q_slot, o_dst, l_dst):
            for r in range(NB):
                rr = slice(r * BQ, (r + 1) * BQ)
                for h in range(HQ):
                    hc = slice(h * DH, (h + 1) * DH)
                    qv = qbuf[q_slot, rr, hc]
                    ks = kp_ref[r, :, :, hc].reshape(KV_R, DH)
                    vs = vp_ref[r, :, :, hc].reshape(KV_R, DH)
                    s = lax.dot_general(
                        qv, ks, (((1,), (1,)), ((), ())),
                        preferred_element_type=jnp.float32)
                    e = jnp.exp(s)
                    o_dst[rr, hc] = lax.dot_general(
                        e.astype(jnp.bfloat16), vs, (((1,), (0,)), ((), ())),
                        preferred_element_type=jnp.float32)
                    l_dst[rr, h:h + 1] = jnp.sum(e, axis=1, keepdims=True)

        partial(0, pme_o, pme_l)

        qsend[0].wait_recv()
        qsend[1].start()
        partial(1, olbuf.at[0], lbuf.at[0])
        osend[0].start()
        lsend[0].start()

        qsend[1].wait_recv()
        qsend[2].start()
        partial(2, ptmp_o, ptmp_l)
        osend[0].wait_recv()
        lsend[0].wait_recv()
        olbuf[1, :, :] = olbuf[1, :, :] + ptmp_o[:, :]
        lbuf[1, :, :] = lbuf[1, :, :] + ptmp_l[:, :]
        osend[1].start()
        lsend[1].start()

        qsend[2].wait_recv()
        partial(3, ptmp_o, ptmp_l)
        osend[1].wait_recv()
        lsend[1].wait_recv()
        olbuf[2, :, :] = olbuf[2, :, :] + ptmp_o[:, :]
        lbuf[2, :, :] = lbuf[2, :, :] + ptmp_l[:, :]
        osend[2].start()
        lsend[2].start()

        osend[2].wait_recv()
        lsend[2].wait_recv()
        o_sum = olbuf[3, :, :] + pme_o[:, :]
        l_sum = lbuf[3, :, :] + pme_l[:, :]
        rows = []
        for r in range(NB):
            rr = slice(r * BQ, (r + 1) * BQ)
            blocks = [o_sum[rr, h * DH:(h + 1) * DH] / l_sum[rr, h:h + 1]
                      for h in range(HQ)]
            rows.append(jnp.concatenate(blocks, axis=1))
        ctx = jnp.concatenate(rows, axis=0)
        out_ref[...] = jnp.dot(ctx.astype(jnp.bfloat16),
                               wo_ref[...].astype(jnp.bfloat16),
                               preferred_element_type=jnp.float32)

        for k in range(N_DEV - 1):
            qsend[k].wait_send()
            osend[k].wait_send()
            lsend[k].wait_send()

    out = pl.pallas_call(
        body,
        out_shape=jax.ShapeDtypeStruct((SQ, D_MODEL), jnp.float32),
        in_specs=[
            pl.BlockSpec(memory_space=pltpu.VMEM),
            pl.BlockSpec(memory_space=pltpu.VMEM),
            pl.BlockSpec(memory_space=pl.ANY),
            pl.BlockSpec(memory_space=pl.ANY),
            pl.BlockSpec(memory_space=pltpu.VMEM),
        ],
        out_specs=pl.BlockSpec(memory_space=pltpu.VMEM),
        scratch_shapes=[
            pltpu.VMEM((N_DEV, SQ, D_MODEL), jnp.bfloat16),
            pltpu.VMEM((N_DEV, SQ, D_MODEL), jnp.float32),
            pltpu.VMEM((N_DEV, SQ, HQ), jnp.float32),
            pltpu.VMEM((SQ, D_MODEL), jnp.float32),
            pltpu.VMEM((SQ, HQ), jnp.float32),
            pltpu.VMEM((SQ, D_MODEL), jnp.float32),
            pltpu.VMEM((SQ, HQ), jnp.float32),
            pltpu.VMEM((NB, GK, BQ, D_MODEL), jnp.bfloat16),
            pltpu.VMEM((NB, GK, BQ, D_MODEL), jnp.bfloat16),
            pltpu.SemaphoreType.DMA((N_DEV - 1,)),
            pltpu.SemaphoreType.DMA((N_DEV - 1,)),
            pltpu.SemaphoreType.DMA((N_DEV - 1,)),
            pltpu.SemaphoreType.DMA((N_DEV - 1,)),
            pltpu.SemaphoreType.DMA((N_DEV - 1,)),
            pltpu.SemaphoreType.DMA((N_DEV - 1,)),
            pltpu.SemaphoreType.DMA((2 * NB,)),
        ],
        compiler_params=pltpu.CompilerParams(
            collective_id=0, vmem_limit_bytes=60 * 1024 * 1024),
    )(x2, Wq, K4, V4, Wo)
    return out.reshape(1, SQ, D_MODEL)


# device time: 100623 ns/iter; 1.0351x vs baseline; 1.0351x over previous
import jax
import jax.numpy as jnp
from jax import lax
from jax.experimental import pallas as pl
from jax.experimental.pallas import tpu as pltpu

N_DEV = 4
SCALE = 0.08838834764831843
SQ = 256
SKV = 4096
HQ = 8
DH = 128
NB = 4
BQ = 64
GK = SKV // (NB * BQ)
KV_R = GK * BQ
D_MODEL = HQ * DH


def kernel(x, Wq, K_ext, V_ext, Wo):
    x2 = x.reshape(SQ, D_MODEL)
    K4 = K_ext.reshape(GK, NB, BQ, D_MODEL).astype(jnp.bfloat16)
    V4 = V_ext.reshape(GK, NB, BQ, D_MODEL).astype(jnp.bfloat16)

    def body(x_ref, wq_ref, k_ref, v_ref, wo_ref, out_ref,
             qbuf, olbuf, lbuf, pme_o, pme_l, ptmp_o, ptmp_l,
             kp_ref, vp_ref,
             qs_sems, qr_sems, os_sems, or_sems, ls_sems, lr_sems,
             kv_sems):
        my = lax.axis_index("i")
        left = (my + N_DEV - 1) % N_DEV
        right = (my + 1) % N_DEV

        kv_copies = []
        for r in range(NB):
            c = pltpu.make_async_copy(k_ref.at[:, r], kp_ref.at[r],
                                      kv_sems.at[r])
            c.start()
            kv_copies.append(c)
            c = pltpu.make_async_copy(v_ref.at[:, r], vp_ref.at[r],
                                      kv_sems.at[NB + r])
            c.start()
            kv_copies.append(c)

        barrier_sem = pltpu.get_barrier_semaphore()
        for nbr in (left, right):
            pl.semaphore_signal(barrier_sem, inc=1, device_id=(nbr,),
                                device_id_type=pl.DeviceIdType.MESH)
        pl.semaphore_wait(barrier_sem, 2)

        def ring(buf, ssems, rsems, k):
            return pltpu.make_async_remote_copy(
                src_ref=buf.at[k], dst_ref=buf.at[k + 1],
                send_sem=ssems.at[k], recv_sem=rsems.at[k],
                device_id=(right,), device_id_type=pl.DeviceIdType.MESH)

        qsend = [ring(qbuf, qs_sems, qr_sems, k) for k in range(N_DEV - 1)]
        osend = [ring(olbuf, os_sems, or_sems, k) for k in range(N_DEV - 1)]
        lsend = [ring(lbuf, ls_sems, lr_sems, k) for k in range(N_DEV - 1)]

        q = jnp.dot(x_ref[...].astype(jnp.bfloat16),
                    wq_ref[...].astype(jnp.bfloat16),
                    preferred_element_type=jnp.float32) * SCALE
        qbuf[0, :, :] = q.astype(jnp.bfloat16)
        qsend[0].start()

        for c in kv_copies:
            c.wait()

        def partial(q_slot, o_dst, l_dst):
            for h in range(HQ):
                hc = slice(h * DH, (h + 1) * DH)
                qv = qbuf[q_slot, :, hc].reshape(NB, BQ, DH)
                ks = kp_ref[:, :, :, hc].reshape(NB, KV_R, DH)
                vs = vp_ref[:, :, :, hc].reshape(NB, KV_R, DH)
                s = lax.dot_general(
                    qv, ks, (((2,), (2,)), ((0,), (0,))),
                    preferred_element_type=jnp.float32)
                e = jnp.exp(s)
                o = lax.dot_general(
                    e.astype(jnp.bfloat16), vs, (((2,), (1,)), ((0,), (0,))),
                    preferred_element_type=jnp.float32)
                o_dst[:, hc] = o.reshape(SQ, DH)
                l_dst[:, h:h + 1] = jnp.sum(e, axis=2).reshape(SQ, 1)

        partial(0, pme_o, pme_l)

        qsend[0].wait_recv()
        qsend[1].start()
        partial(1, olbuf.at[0], lbuf.at[0])
        osend[0].start()
        lsend[0].start()

        qsend[1].wait_recv()
        qsend[2].start()
        partial(2, ptmp_o, ptmp_l)
        osend[0].wait_recv()
        lsend[0].wait_recv()
        olbuf[1, :, :] = olbuf[1, :, :] + ptmp_o[:, :]
        lbuf[1, :, :] = lbuf[1, :, :] + ptmp_l[:, :]
        osend[1].start()
        lsend[1].start()

        qsend[2].wait_recv()
        partial(3, ptmp_o, ptmp_l)
        osend[1].wait_recv()
        lsend[1].wait_recv()
        olbuf[2, :, :] = olbuf[2, :, :] + ptmp_o[:, :]
        lbuf[2, :, :] = lbuf[2, :, :] + ptmp_l[:, :]
        osend[2].start()
        lsend[2].start()

        osend[2].wait_recv()
        lsend[2].wait_recv()
        o_sum = olbuf[3, :, :] + pme_o[:, :]
        l_sum = lbuf[3, :, :] + pme_l[:, :]
        ctx = jnp.concatenate(
            [o_sum[:, h * DH:(h + 1) * DH] / l_sum[:, h:h + 1]
             for h in range(HQ)], axis=1)
        out_ref[...] = jnp.dot(ctx.astype(jnp.bfloat16),
                               wo_ref[...].astype(jnp.bfloat16),
                               preferred_element_type=jnp.float32)

        for k in range(N_DEV - 1):
            qsend[k].wait_send()
            osend[k].wait_send()
            lsend[k].wait_send()

    out = pl.pallas_call(
        body,
        out_shape=jax.ShapeDtypeStruct((SQ, D_MODEL), jnp.float32),
        in_specs=[
            pl.BlockSpec(memory_space=pltpu.VMEM),
            pl.BlockSpec(memory_space=pltpu.VMEM),
            pl.BlockSpec(memory_space=pl.ANY),
            pl.BlockSpec(memory_space=pl.ANY),
            pl.BlockSpec(memory_space=pltpu.VMEM),
        ],
        out_specs=pl.BlockSpec(memory_space=pltpu.VMEM),
        scratch_shapes=[
            pltpu.VMEM((N_DEV, SQ, D_MODEL), jnp.bfloat16),
            pltpu.VMEM((N_DEV, SQ, D_MODEL), jnp.float32),
            pltpu.VMEM((N_DEV, SQ, HQ), jnp.float32),
            pltpu.VMEM((SQ, D_MODEL), jnp.float32),
            pltpu.VMEM((SQ, HQ), jnp.float32),
            pltpu.VMEM((SQ, D_MODEL), jnp.float32),
            pltpu.VMEM((SQ, HQ), jnp.float32),
            pltpu.VMEM((NB, GK, BQ, D_MODEL), jnp.bfloat16),
            pltpu.VMEM((NB, GK, BQ, D_MODEL), jnp.bfloat16),
            pltpu.SemaphoreType.DMA((N_DEV - 1,)),
            pltpu.SemaphoreType.DMA((N_DEV - 1,)),
            pltpu.SemaphoreType.DMA((N_DEV - 1,)),
            pltpu.SemaphoreType.DMA((N_DEV - 1,)),
            pltpu.SemaphoreType.DMA((N_DEV - 1,)),
            pltpu.SemaphoreType.DMA((N_DEV - 1,)),
            pltpu.SemaphoreType.DMA((2 * NB,)),
        ],
        compiler_params=pltpu.CompilerParams(
            collective_id=0, vmem_limit_bytes=60 * 1024 * 1024),
    )(x2, Wq, K4, V4, Wo)
    return out.reshape(1, SQ, D_MODEL)


# device time: 83124 ns/iter; 1.2530x vs baseline; 1.2105x over previous
import jax
import jax.numpy as jnp
from jax import lax
from jax.experimental import pallas as pl
from jax.experimental.pallas import tpu as pltpu

N_DEV = 4
SCALE = 0.08838834764831843
SQ = 256
SKV = 4096
HQ = 8
DH = 128
NB = 4
BQ = 64
GK = SKV // (NB * BQ)
KV_R = GK * BQ
D_MODEL = HQ * DH


def kernel(x, Wq, K_ext, V_ext, Wo):
    x2 = x.reshape(SQ, D_MODEL)
    K4 = K_ext.reshape(GK, NB, BQ, D_MODEL).astype(jnp.bfloat16)
    V4 = V_ext.reshape(GK, NB, BQ, D_MODEL).astype(jnp.bfloat16)

    def body(x_ref, wq_ref, k_ref, v_ref, wo_ref, out_ref,
             qbuf, olbuf, lbuf, pme_o, pme_l, ptmp_o, ptmp_l,
             kp_ref, vp_ref,
             qs_sems, qr_sems, os_sems, or_sems, ls_sems, lr_sems,
             kv_sems):
        my = lax.axis_index("i")
        left = (my + N_DEV - 1) % N_DEV
        right = (my + 1) % N_DEV

        kv_copies = []
        for r in range(NB):
            c = pltpu.make_async_copy(k_ref.at[:, r], kp_ref.at[r],
                                      kv_sems.at[r])
            c.start()
            kv_copies.append(c)
            c = pltpu.make_async_copy(v_ref.at[:, r], vp_ref.at[r],
                                      kv_sems.at[NB + r])
            c.start()
            kv_copies.append(c)

        barrier_sem = pltpu.get_barrier_semaphore()
        for nbr in (left, right):
            pl.semaphore_signal(barrier_sem, inc=1, device_id=(nbr,),
                                device_id_type=pl.DeviceIdType.MESH)

        def ring(buf, ssems, rsems, k):
            return pltpu.make_async_remote_copy(
                src_ref=buf.at[k], dst_ref=buf.at[k + 1],
                send_sem=ssems.at[k], recv_sem=rsems.at[k],
                device_id=(right,), device_id_type=pl.DeviceIdType.MESH)

        qsend = [ring(qbuf, qs_sems, qr_sems, k) for k in range(N_DEV - 1)]
        osend = [ring(olbuf, os_sems, or_sems, k) for k in range(N_DEV - 1)]
        lsend = [ring(lbuf, ls_sems, lr_sems, k) for k in range(N_DEV - 1)]

        q = jnp.dot(x_ref[...].astype(jnp.bfloat16),
                    wq_ref[...].astype(jnp.bfloat16),
                    preferred_element_type=jnp.float32) * SCALE
        pl.semaphore_wait(barrier_sem, 2)
        qbuf[0, :, :] = q.astype(jnp.bfloat16)
        qsend[0].start()

        for c in kv_copies:
            c.wait()

        def partial(q_slot, o_dst, l_dst):
            for h in range(HQ):
                hc = slice(h * DH, (h + 1) * DH)
                qv = qbuf[q_slot, :, hc].reshape(NB, BQ, DH)
                ks = kp_ref[:, :, :, hc].reshape(NB, KV_R, DH)
                vs = vp_ref[:, :, :, hc].reshape(NB, KV_R, DH)
                s = lax.dot_general(
                    qv, ks, (((2,), (2,)), ((0,), (0,))),
                    preferred_element_type=jnp.float32)
                e = jnp.exp(s)
                o = lax.dot_general(
                    e.astype(jnp.bfloat16), vs, (((2,), (1,)), ((0,), (0,))),
                    preferred_element_type=jnp.float32)
                o_dst[:, hc] = o.reshape(SQ, DH).astype(o_dst.dtype)
                l_dst[:, h:h + 1] = jnp.sum(e, axis=2).reshape(SQ, 1)

        partial(0, pme_o, pme_l)

        qsend[0].wait_recv()
        qsend[1].start()
        partial(1, olbuf.at[0], lbuf.at[0])
        osend[0].start()
        lsend[0].start()

        qsend[1].wait_recv()
        qsend[2].start()
        partial(2, ptmp_o, ptmp_l)
        osend[0].wait_recv()
        lsend[0].wait_recv()
        olbuf[1, :, :] = (olbuf[1, :, :] + ptmp_o[:, :]).astype(jnp.bfloat16)
        lbuf[1, :, :] = lbuf[1, :, :] + ptmp_l[:, :]
        osend[1].start()
        lsend[1].start()

        qsend[2].wait_recv()
        partial(3, ptmp_o, ptmp_l)
        osend[1].wait_recv()
        lsend[1].wait_recv()
        olbuf[2, :, :] = (olbuf[2, :, :] + ptmp_o[:, :]).astype(jnp.bfloat16)
        lbuf[2, :, :] = lbuf[2, :, :] + ptmp_l[:, :]
        osend[2].start()
        lsend[2].start()

        osend[2].wait_recv()
        lsend[2].wait_recv()
        o_sum = olbuf[3, :, :].astype(jnp.float32) + pme_o[:, :]
        l_sum = lbuf[3, :, :] + pme_l[:, :]
        ctx = jnp.concatenate(
            [o_sum[:, h * DH:(h + 1) * DH] / l_sum[:, h:h + 1]
             for h in range(HQ)], axis=1)
        out_ref[...] = jnp.dot(ctx.astype(jnp.bfloat16),
                               wo_ref[...].astype(jnp.bfloat16),
                               preferred_element_type=jnp.float32)

        for k in range(N_DEV - 1):
            qsend[k].wait_send()
            osend[k].wait_send()
            lsend[k].wait_send()

    out = pl.pallas_call(
        body,
        out_shape=jax.ShapeDtypeStruct((SQ, D_MODEL), jnp.float32),
        in_specs=[
            pl.BlockSpec(memory_space=pltpu.VMEM),
            pl.BlockSpec(memory_space=pltpu.VMEM),
            pl.BlockSpec(memory_space=pl.ANY),
            pl.BlockSpec(memory_space=pl.ANY),
            pl.BlockSpec(memory_space=pltpu.VMEM),
        ],
        out_specs=pl.BlockSpec(memory_space=pltpu.VMEM),
        scratch_shapes=[
            pltpu.VMEM((N_DEV, SQ, D_MODEL), jnp.bfloat16),
            pltpu.VMEM((N_DEV, SQ, D_MODEL), jnp.bfloat16),
            pltpu.VMEM((N_DEV, SQ, HQ), jnp.float32),
            pltpu.VMEM((SQ, D_MODEL), jnp.float32),
            pltpu.VMEM((SQ, HQ), jnp.float32),
            pltpu.VMEM((SQ, D_MODEL), jnp.float32),
            pltpu.VMEM((SQ, HQ), jnp.float32),
            pltpu.VMEM((NB, GK, BQ, D_MODEL), jnp.bfloat16),
            pltpu.VMEM((NB, GK, BQ, D_MODEL), jnp.bfloat16),
            pltpu.SemaphoreType.DMA((N_DEV - 1,)),
            pltpu.SemaphoreType.DMA((N_DEV - 1,)),
            pltpu.SemaphoreType.DMA((N_DEV - 1,)),
            pltpu.SemaphoreType.DMA((N_DEV - 1,)),
            pltpu.SemaphoreType.DMA((N_DEV - 1,)),
            pltpu.SemaphoreType.DMA((N_DEV - 1,)),
            pltpu.SemaphoreType.DMA((2 * NB,)),
        ],
        compiler_params=pltpu.CompilerParams(
            collective_id=0, vmem_limit_bytes=60 * 1024 * 1024),
    )(x2, Wq, K4, V4, Wo)
    return out.reshape(1, SQ, D_MODEL)


# device time: 76297 ns/iter; 1.3651x vs baseline; 1.0895x over previous
import jax
import jax.numpy as jnp
from jax import lax
from jax.experimental import pallas as pl
from jax.experimental.pallas import tpu as pltpu

N_DEV = 4
SCALE = 0.08838834764831843
SQ = 256
SKV = 4096
HQ = 8
DH = 128
NB = 4
BQ = 64
GK = SKV // (NB * BQ)
KV_R = GK * BQ
D_MODEL = HQ * DH


def kernel(x, Wq, K_ext, V_ext, Wo):
    x2 = x.reshape(SQ, D_MODEL)
    K4 = K_ext.reshape(GK, NB, BQ, D_MODEL).astype(jnp.bfloat16)
    V4 = V_ext.reshape(GK, NB, BQ, D_MODEL).astype(jnp.bfloat16)

    def body(x_ref, wq_ref, k_ref, v_ref, wo_ref, out_ref,
             qbuf, psrc, pbuf, lsrc, lbuf, pme_o, pme_l,
             kp_ref, vp_ref,
             qs_sems, qr_sems, ps_sems, pr_sems, ls_sems, lr_sems,
             kv_sems):
        my = lax.axis_index("i")
        left = (my + N_DEV - 1) % N_DEV
        right = (my + 1) % N_DEV
        diag = (my + 2) % N_DEV

        kv_copies = []
        for r in range(NB):
            c = pltpu.make_async_copy(k_ref.at[:, r], kp_ref.at[r],
                                      kv_sems.at[r])
            c.start()
            kv_copies.append(c)
            c = pltpu.make_async_copy(v_ref.at[:, r], vp_ref.at[r],
                                      kv_sems.at[NB + r])
            c.start()
            kv_copies.append(c)

        barrier_sem = pltpu.get_barrier_semaphore()
        for nbr in (left, right, diag):
            pl.semaphore_signal(barrier_sem, inc=1, device_id=(nbr,),
                                device_id_type=pl.DeviceIdType.MESH)

        targets = (right, diag, left)

        def direct(src, dst, ssems, rsems, k):
            return pltpu.make_async_remote_copy(
                src_ref=src, dst_ref=dst,
                send_sem=ssems.at[k], recv_sem=rsems.at[k],
                device_id=(targets[k],),
                device_id_type=pl.DeviceIdType.MESH)

        qsend = [direct(qbuf.at[0], qbuf.at[k + 1], qs_sems, qr_sems, k)
                 for k in range(3)]
        psend = [direct(psrc.at[k], pbuf.at[k], ps_sems, pr_sems, k)
                 for k in range(3)]
        lsend = [direct(lsrc.at[k], lbuf.at[k], ls_sems, lr_sems, k)
                 for k in range(3)]

        q = jnp.dot(x_ref[...].astype(jnp.bfloat16),
                    wq_ref[...].astype(jnp.bfloat16),
                    preferred_element_type=jnp.float32) * SCALE
        pl.semaphore_wait(barrier_sem, 3)
        qbuf[0, :, :] = q.astype(jnp.bfloat16)
        for k in range(3):
            qsend[k].start()

        for c in kv_copies:
            c.wait()

        def partial(q_slot, o_dst, l_dst, r0=0, r1=NB):
            nr = r1 - r0
            rows = slice(r0 * BQ, r1 * BQ)
            for h in range(HQ):
                hc = slice(h * DH, (h + 1) * DH)
                qv = qbuf[q_slot, rows, hc].reshape(nr, BQ, DH)
                ks = kp_ref[r0:r1, :, :, hc].reshape(nr, KV_R, DH)
                vs = vp_ref[r0:r1, :, :, hc].reshape(nr, KV_R, DH)
                s = lax.dot_general(
                    qv, ks, (((2,), (2,)), ((0,), (0,))),
                    preferred_element_type=jnp.float32)
                e = jnp.exp(s)
                o = lax.dot_general(
                    e.astype(jnp.bfloat16), vs, (((2,), (1,)), ((0,), (0,))),
                    preferred_element_type=jnp.float32)
                o_dst[rows, hc] = o.reshape(nr * BQ, DH).astype(o_dst.dtype)
                l_dst[rows, h:h + 1] = jnp.sum(e, axis=2).reshape(nr * BQ, 1)

        partial(0, pme_o, pme_l, 0, NB // 2)

        for k, q_slot in enumerate((3, 2, 1)):
            qsend[2 - k].wait_recv()
            partial(q_slot, psrc.at[k], lsrc.at[k])
            psend[k].start()
            lsend[k].start()

        partial(0, pme_o, pme_l, NB // 2, NB)

        psend[0].wait_recv()
        lsend[0].wait_recv()
        psend[1].wait_recv()
        lsend[1].wait_recv()
        o01 = (pme_o[:, :] + pbuf[0, :, :].astype(jnp.float32)
               + pbuf[1, :, :].astype(jnp.float32))
        l01 = pme_l[:, :] + lbuf[0, :, :] + lbuf[1, :, :]
        psend[2].wait_recv()
        lsend[2].wait_recv()
        o_sum = o01 + pbuf[2, :, :].astype(jnp.float32)
        l_sum = l01 + lbuf[2, :, :]
        ctx = jnp.concatenate(
            [o_sum[:, h * DH:(h + 1) * DH] / l_sum[:, h:h + 1]
             for h in range(HQ)], axis=1)
        out_ref[...] = jnp.dot(ctx.astype(jnp.bfloat16),
                               wo_ref[...].astype(jnp.bfloat16),
                               preferred_element_type=jnp.float32)

        for k in range(3):
            qsend[k].wait_send()
            psend[k].wait_send()
            lsend[k].wait_send()

    out = pl.pallas_call(
        body,
        out_shape=jax.ShapeDtypeStruct((SQ, D_MODEL), jnp.float32),
        in_specs=[
            pl.BlockSpec(memory_space=pltpu.VMEM),
            pl.BlockSpec(memory_space=pltpu.VMEM),
            pl.BlockSpec(memory_space=pl.ANY),
            pl.BlockSpec(memory_space=pl.ANY),
            pl.BlockSpec(memory_space=pltpu.VMEM),
        ],
        out_specs=pl.BlockSpec(memory_space=pltpu.VMEM),
        scratch_shapes=[
            pltpu.VMEM((N_DEV, SQ, D_MODEL), jnp.bfloat16),
            pltpu.VMEM((3, SQ, D_MODEL), jnp.bfloat16),
            pltpu.VMEM((3, SQ, D_MODEL), jnp.bfloat16),
            pltpu.VMEM((3, SQ, HQ), jnp.float32),
            pltpu.VMEM((3, SQ, HQ), jnp.float32),
            pltpu.VMEM((SQ, D_MODEL), jnp.float32),
            pltpu.VMEM((SQ, HQ), jnp.float32),
            pltpu.VMEM((NB, GK, BQ, D_MODEL), jnp.bfloat16),
            pltpu.VMEM((NB, GK, BQ, D_MODEL), jnp.bfloat16),
            pltpu.SemaphoreType.DMA((3,)),
            pltpu.SemaphoreType.DMA((3,)),
            pltpu.SemaphoreType.DMA((3,)),
            pltpu.SemaphoreType.DMA((3,)),
            pltpu.SemaphoreType.DMA((3,)),
            pltpu.SemaphoreType.DMA((3,)),
            pltpu.SemaphoreType.DMA((2 * NB,)),
        ],
        compiler_params=pltpu.CompilerParams(
            collective_id=0, vmem_limit_bytes=60 * 1024 * 1024),
    )(x2, Wq, K4, V4, Wo)
    return out.reshape(1, SQ, D_MODEL)
